# Initial kernel scaffold; baseline (speedup 1.0000x reference)
import jax
import jax.numpy as jnp
from jax.experimental import pallas as pl


def kernel(x, tables):
    raise NotImplementedError("write your pallas kernel here")


# speedup vs baseline: 219.7781x; 219.7781x over previous
import functools

import jax
import jax.numpy as jnp
import numpy as np
from jax import lax
from jax.experimental import pallas as pl
from jax.experimental.pallas import tpu as pltpu
from jax.experimental.pallas import tpu_sc as plsc

_DIM = 3
_N_LEVELS = 16
_N_FEATS = 2
_T = 32768
_RANGE = 1.0
_BASE_RES = 16
_FINEST_RES = 512
_growth = np.exp((np.log(_FINEST_RES) - np.log(_BASE_RES)) / (_N_LEVELS - 1))
_RES = np.floor(_BASE_RES * (_growth ** np.arange(_N_LEVELS))).astype(np.float64)
_P1 = np.int32(np.uint32(2654435761))
_P2 = np.int32(np.uint32(805459861))
_MASK = np.int32(_T - 1)

_NC = 2
_NS = 16
_B = 2048
_BS = _B // _NS


def _body(x_hbm, tab_hbm, out_hbm,
          table_v, coord_v, f_v, loc_v, outb_v,
          sem_coord0, sem_coord1, sem_pub, sem_rb, sem_out0, sem_out1,
          shared):
    sem_coord = (sem_coord0, sem_coord1)
    sem_out = (sem_out0, sem_out1)
    c = lax.axis_index("c")
    s = lax.axis_index("s")
    n_pts = x_hbm.shape[0] // _DIM
    n_half = n_pts // _NC
    rounds = n_half // _B

    lanes = lax.iota(jnp.int32, 16)

    chunk = _BS * 32
    rows_per_level = 2 * _T // chunk
    lanes2 = lanes * 2
    zero16 = jnp.zeros((16,), jnp.int32)

    def stage_body(k, carry):
        pltpu.sync_copy(tab_hbm.at[s, pl.ds(k * chunk, chunk)],
                        outb_v.at[pl.ds(0, chunk)])

        def de_body(g, carry2):
            ix = lanes2 + g * 32
            table_v[pl.ds(k * (chunk // 2) + g * 16, 16)] = (
                plsc.load_gather(outb_v, [ix]))
            table_v[pl.ds(_T + k * (chunk // 2) + g * 16, 16)] = (
                plsc.load_gather(outb_v, [ix + 1]))
            return carry2

        lax.fori_loop(0, chunk // 32, de_body, 0, unroll=2)
        return carry

    lax.fori_loop(0, rows_per_level, stage_body, 0)
    resh_s = jnp.float32(0.5 * float(_RES[0]))
    for k in range(1, _N_LEVELS):
        resh_s = jnp.where(s == k, jnp.float32(0.5 * float(_RES[k])), resh_s)
    resh = jnp.full((16,), resh_s, jnp.float32)
    half = jnp.full((16,), 0.5, jnp.float32)
    one = jnp.full((16,), 1.0, jnp.float32)

    l_lo = lanes >> 1
    l_hi = l_lo + 8
    f_ix = lanes & 1

    my_ds = pl.ds(s * _BS, _BS)

    def coord_copies(r, buf):
        base = c * n_half + r * _B
        return [
            pltpu.make_async_copy(
                x_hbm.at[pl.ds(d * n_pts + base, _B)],
                coord_v.at[pl.ds((buf * _DIM + d) * _B, _B)], sem_coord[buf])
            for d in range(_DIM)
        ]

    def out_copy(r, buf):
        row = (c * n_half + r * _B + s * _BS) * 32
        return pltpu.make_async_copy(
            outb_v.at[pl.ds(buf * chunk, chunk)],
            out_hbm.at[pl.ds(row, chunk)], sem_out[buf])

    def transpose_round(buf):
        bo = buf * chunk

        @plsc.parallel_loop(0, _BS, unroll=2)
        def tr_body(t):
            t_ix = jnp.full((16,), t, jnp.int32)
            lo = plsc.load_gather(loc_v, [l_lo, f_ix, t_ix])
            hi = plsc.load_gather(loc_v, [l_hi, f_ix, t_ix])
            outb_v[pl.ds(bo + t * 32, 16)] = lo
            outb_v[pl.ds(bo + t * 32 + 16, 16)] = hi

    def half_round(r, p):
        for cc in coord_copies(r, p):
            cc.start()
        for cc in coord_copies(r, p):
            cc.wait()

        p3 = p * _DIM * _B

        @plsc.parallel_loop(0, _B // 16, unroll=2)
        def group_body(g):
            off = g * 16
            offc = p3 + off
            xr = coord_v[pl.ds(offc, 16)]
            yr = coord_v[pl.ds(offc + _B, 16)]
            zr = coord_v[pl.ds(offc + 2 * _B, 16)]
            xl = (xr + one) * resh
            yl = (yr + one) * resh
            zl = (zr + one) * resh
            xi = xl.astype(jnp.int32)
            yi = yl.astype(jnp.int32)
            zi = zl.astype(jnp.int32)
            fx = xl - xi.astype(jnp.float32)
            fy = yl - yi.astype(jnp.float32)
            fz = zl - zi.astype(jnp.float32)

            ax = xi
            ay = yi * _P1
            az = zi * _P2
            bx = ax + 1
            by = ay + _P1
            bz = az + _P2
            m00 = ay ^ az
            m01 = ay ^ bz
            m10 = by ^ az
            m11 = by ^ bz

            gx0 = one - fx
            gy0 = one - fy
            gz0 = one - fz
            w00 = gy0 * gz0
            w01 = gy0 * fz
            w10 = fy * gz0
            w11 = fy * fz

            corners = (
                (ax, m00, gx0, w00), (bx, m00, fx, w00),
                (ax, m10, gx0, w10), (bx, m10, fx, w10),
                (ax, m01, gx0, w01), (bx, m01, fx, w01),
                (ax, m11, gx0, w11), (bx, m11, fx, w11),
            )
            idxs = [(hx ^ myz) & _MASK for hx, myz, _, _ in corners]
            g0s = [plsc.load_gather(table_v, [i]) for i in idxs]
            g1s = [plsc.load_gather(table_v, [i + _T]) for i in idxs]
            ws = [wx * wyz for _, _, wx, wyz in corners]

            def tree_sum(vals):
                while len(vals) > 1:
                    vals = [a + b for a, b in zip(vals[::2], vals[1::2])]
                return vals[0]

            f_v[0, pl.ds(off, 16)] = tree_sum(
                [w * g for w, g in zip(ws, g0s)])
            f_v[1, pl.ds(off, 16)] = tree_sum(
                [w * g for w, g in zip(ws, g1s)])

        pltpu.make_async_copy(f_v, shared.at[p, s], sem_pub).start()
        pltpu.make_async_copy(f_v, shared.at[p, s], sem_pub).wait()
        plsc.subcore_barrier()
        pltpu.make_async_copy(shared.at[p, :, :, my_ds], loc_v, sem_rb).start()
        pltpu.make_async_copy(shared.at[p, :, :, my_ds], loc_v, sem_rb).wait()
        transpose_round(p)
        out_copy(r, p).start()
        out_copy(r, p).wait()

    def pair_body(rp, carry):
        half_round(2 * rp, 0)
        half_round(2 * rp + 1, 1)
        return carry

    lax.fori_loop(0, rounds // 2, pair_body, 0)


@jax.jit
def _hash_grid_encode(x, tabs):
    n = x.shape[0] // _DIM
    mesh = plsc.VectorSubcoreMesh(core_axis_name="c", subcore_axis_name="s")
    fn = pl.kernel(
        _body,
        out_type=jax.ShapeDtypeStruct((n * _N_LEVELS * _N_FEATS,), jnp.float32),
        mesh=mesh,
        compiler_params=pltpu.CompilerParams(needs_layout_passes=False),
        scratch_types=[
            pltpu.VMEM((2 * _T,), jnp.float32),
            pltpu.VMEM((2 * _DIM * _B,), jnp.float32),
            pltpu.VMEM((_N_FEATS, _B), jnp.float32),
            pltpu.VMEM((_NS, _N_FEATS, _BS), jnp.float32),
            pltpu.VMEM((2 * _BS * 32,), jnp.float32),
            pltpu.SemaphoreType.DMA,
            pltpu.SemaphoreType.DMA,
            pltpu.SemaphoreType.DMA,
            pltpu.SemaphoreType.DMA,
            pltpu.SemaphoreType.DMA,
            pltpu.SemaphoreType.DMA,
            pltpu.VMEM_SHARED((2, _NS, _N_FEATS, _B), jnp.float32),
        ],
    )
    return fn(x, tabs)


def kernel(x, tables):
    n = x.shape[0]
    tabs = tables.reshape(_N_LEVELS, _N_FEATS * _T)
    out = _hash_grid_encode(x.T.reshape(-1), tabs)
    return out.reshape(n, _N_LEVELS * _N_FEATS)


# speedup vs baseline: 221.3972x; 1.0074x over previous
import functools

import jax
import jax.numpy as jnp
import numpy as np
from jax import lax
from jax.experimental import pallas as pl
from jax.experimental.pallas import tpu as pltpu
from jax.experimental.pallas import tpu_sc as plsc

_DIM = 3
_N_LEVELS = 16
_N_FEATS = 2
_T = 32768
_RANGE = 1.0
_BASE_RES = 16
_FINEST_RES = 512
_growth = np.exp((np.log(_FINEST_RES) - np.log(_BASE_RES)) / (_N_LEVELS - 1))
_RES = np.floor(_BASE_RES * (_growth ** np.arange(_N_LEVELS))).astype(np.float64)
_P1 = np.int32(np.uint32(2654435761))
_P2 = np.int32(np.uint32(805459861))
_MASK = np.int32(_T - 1)

_NC = 2
_NS = 16
_B = 4096
_BS = _B // _NS


def _body(x_hbm, tab_hbm, out_hbm,
          table_v, coord_v, f_v, loc_v, outb_v, shared):
    c = lax.axis_index("c")
    s = lax.axis_index("s")
    n_half = x_hbm.shape[1] // _NC
    rounds = n_half // _B

    lanes = lax.iota(jnp.int32, 16)

    tmp_v = outb_v
    chunk = tmp_v.shape[0]
    lanes2 = lanes * 2

    def stage_body(k, carry):
        pltpu.sync_copy(tab_hbm.at[s, pl.ds(k * chunk, chunk)], tmp_v)

        def de_body(g, carry2):
            ix = lanes2 + g * 32
            table_v[pl.ds(k * (chunk // 2) + g * 16, 16)] = (
                plsc.load_gather(tmp_v, [ix]))
            table_v[pl.ds(_T + k * (chunk // 2) + g * 16, 16)] = (
                plsc.load_gather(tmp_v, [ix + 1]))
            return carry2

        lax.fori_loop(0, chunk // 32, de_body, 0, unroll=2)
        return carry

    lax.fori_loop(0, 2 * _T // chunk, stage_body, 0)
    resh_s = jnp.float32(0.5 * float(_RES[0]))
    for k in range(1, _N_LEVELS):
        resh_s = jnp.where(s == k, jnp.float32(0.5 * float(_RES[k])), resh_s)
    resh = jnp.full((16,), resh_s, jnp.float32)
    half = jnp.full((16,), 0.5, jnp.float32)
    one = jnp.full((16,), 1.0, jnp.float32)

    l_lo = lanes >> 1
    l_hi = l_lo + 8
    f_ix = lanes & 1

    def round_body(r, carry):
        base = c * n_half + r * _B

        pltpu.sync_copy(x_hbm.at[:, pl.ds(base, _B)], coord_v)

        @plsc.parallel_loop(0, _B // 16, unroll=2)
        def group_body(g):
            off = g * 16
            xr = coord_v[0, pl.ds(off, 16)]
            yr = coord_v[1, pl.ds(off, 16)]
            zr = coord_v[2, pl.ds(off, 16)]
            xl = (xr + one) * resh
            yl = (yr + one) * resh
            zl = (zr + one) * resh
            xi = xl.astype(jnp.int32)
            yi = yl.astype(jnp.int32)
            zi = zl.astype(jnp.int32)
            fx = xl - xi.astype(jnp.float32)
            fy = yl - yi.astype(jnp.float32)
            fz = zl - zi.astype(jnp.float32)

            ax = xi
            ay = yi * _P1
            az = zi * _P2
            bx = ax + 1
            by = ay + _P1
            bz = az + _P2
            m00 = ay ^ az
            m01 = ay ^ bz
            m10 = by ^ az
            m11 = by ^ bz

            gx0 = one - fx
            gy0 = one - fy
            gz0 = one - fz
            w00 = gy0 * gz0
            w01 = gy0 * fz
            w10 = fy * gz0
            w11 = fy * fz

            corners = (
                (ax, m00, gx0, w00), (bx, m00, fx, w00),
                (ax, m10, gx0, w10), (bx, m10, fx, w10),
                (ax, m01, gx0, w01), (bx, m01, fx, w01),
                (ax, m11, gx0, w11), (bx, m11, fx, w11),
            )
            idxs = [(hx ^ myz) & _MASK for hx, myz, _, _ in corners]
            g0s = [plsc.load_gather(table_v, [i]) for i in idxs]
            g1s = [plsc.load_gather(table_v, [i + _T]) for i in idxs]
            ws = [wx * wyz for _, _, wx, wyz in corners]

            def tree_sum(vals):
                while len(vals) > 1:
                    vals = [a + b for a, b in zip(vals[::2], vals[1::2])]
                return vals[0]

            f_v[0, pl.ds(off, 16)] = tree_sum(
                [w * g for w, g in zip(ws, g0s)])
            f_v[1, pl.ds(off, 16)] = tree_sum(
                [w * g for w, g in zip(ws, g1s)])

        buf = r % 2
        pltpu.sync_copy(f_v, shared.at[buf, s])
        plsc.subcore_barrier()

        pltpu.sync_copy(shared.at[buf, :, :, pl.ds(s * _BS, _BS)], loc_v)

        def tr_body(p, carry2):
            p_ix = jnp.full((16,), p, jnp.int32)
            lo = plsc.load_gather(loc_v, [l_lo, f_ix, p_ix])
            hi = plsc.load_gather(loc_v, [l_hi, f_ix, p_ix])
            outb_v[pl.ds(p * 32, 16)] = lo
            outb_v[pl.ds(p * 32 + 16, 16)] = hi
            return carry2

        lax.fori_loop(0, _BS, tr_body, 0, unroll=2)

        row = base + s * _BS
        pltpu.sync_copy(outb_v, out_hbm.at[pl.ds(row * 32, _BS * 32)])
        return carry

    lax.fori_loop(0, rounds, round_body, 0)


@jax.jit
def _hash_grid_encode(x, tabs):
    n = x.shape[1]
    mesh = plsc.VectorSubcoreMesh(core_axis_name="c", subcore_axis_name="s")
    fn = pl.kernel(
        _body,
        out_type=jax.ShapeDtypeStruct((n * _N_LEVELS * _N_FEATS,), jnp.float32),
        mesh=mesh,
        compiler_params=pltpu.CompilerParams(needs_layout_passes=False),
        scratch_types=[
            pltpu.VMEM((2 * _T,), jnp.float32),
            pltpu.VMEM((_DIM, _B), jnp.float32),
            pltpu.VMEM((_N_FEATS, _B), jnp.float32),
            pltpu.VMEM((_NS, _N_FEATS, _BS), jnp.float32),
            pltpu.VMEM((_BS * 32,), jnp.float32),
            pltpu.VMEM_SHARED((2, _NS, _N_FEATS, _B), jnp.float32),
        ],
    )
    return fn(x, tabs)


def kernel(x, tables):
    n = x.shape[0]
    tabs = tables.reshape(_N_LEVELS, _N_FEATS * _T)
    out = _hash_grid_encode(x.T, tabs)
    return out.reshape(n, _N_LEVELS * _N_FEATS)
